# Initial kernel scaffold; baseline (speedup 1.0000x reference)
import jax
import jax.numpy as jnp
from jax.experimental import pallas as pl


def kernel(input_features, fg_w1, fg_b1, fg_w2):
    raise NotImplementedError("write your pallas kernel here")


# speedup vs baseline: 3.9478x; 3.9478x over previous
import numpy as np
import jax
import jax.numpy as jnp
from jax.experimental import pallas as pl

_B, _L, _C, _E = 2, 2048, 2048, 8
_K = 100
_LT = 256
_CT = 512
_HI = float(np.float32(1.0) + np.float32(1e-4))
_LO = float(np.float32(1e-4))


def _pos_code():
    half = _C // 2
    theta = 12000.0 ** (-jnp.arange(0, half, dtype=jnp.float32) / half)
    freqs = jnp.einsum('i,j->ij', jnp.arange(_L, dtype=jnp.float32), theta)
    sin = jnp.sin(freqs)
    cos = jnp.cos(freqs)
    return jnp.concatenate([cos - sin, sin + cos], axis=-1)


def _gumbel():
    u = jax.random.uniform(jax.random.key(42), (_B, _L, _E),
                           minval=1e-10, maxval=1.0)
    return -jnp.log(-jnp.log(u))


def _mm1_body(pos_ref, w1_ref, b1_ref, out_ref):
    out_ref[...] = jnp.dot(pos_ref[...], w1_ref[...],
                           preferred_element_type=jnp.float32) + b1_ref[...]


def _route_body(h1_ref, w2_ref, gt_ref, p_ref):
    lg = jnp.dot(h1_ref[...], w2_ref[...],
                 preferred_element_type=jnp.float32)
    lt = jnp.transpose(lg[:, :_E])
    eidx = jax.lax.broadcasted_iota(jnp.int32, (_E, _L), 0)
    for b in range(_B):
        z = lt + gt_ref[b]
        zmax = jnp.max(z, axis=0, keepdims=True)
        first = jnp.min(jnp.where(z == zmax, eidx, _E), axis=0,
                        keepdims=True)
        onehot = eidx == first
        cnt = jnp.sum(onehot.astype(jnp.int32), axis=1, keepdims=True)
        small = jnp.where(cnt >= _K, 0.0, _LO).astype(jnp.float32)
        p_ref[b] = jnp.where(onehot, _HI, small)


def _scale_body(s_ref, x_ref, out_ref):
    st = jnp.transpose(s_ref[0])
    out_ref[...] = x_ref[...] * st[None]


def kernel(input_features, fg_w1, fg_b1, fg_w2):
    pos = _pos_code()
    gt = jnp.transpose(_gumbel(), (0, 2, 1))

    pre = pl.pallas_call(
        _mm1_body,
        grid=(_L // _LT,),
        in_specs=[
            pl.BlockSpec((_LT, _C), lambda i: (i, 0)),
            pl.BlockSpec((_C, _C), lambda i: (0, 0)),
            pl.BlockSpec((1, _C), lambda i: (0, 0)),
        ],
        out_specs=pl.BlockSpec((_LT, _C), lambda i: (i, 0)),
        out_shape=jax.ShapeDtypeStruct((_L, _C), jnp.float32),
    )(pos, fg_w1, fg_b1.reshape(1, _C))

    h1 = jax.nn.gelu(pre, approximate=False)

    P = pl.pallas_call(
        _route_body,
        in_specs=[
            pl.BlockSpec((_L, _C), lambda: (0, 0)),
            pl.BlockSpec((_C, 2 * _E), lambda: (0, 0)),
            pl.BlockSpec((_B, _E, _L), lambda: (0, 0, 0)),
        ],
        out_specs=pl.BlockSpec((_B, _E, _L), lambda: (0, 0, 0)),
        out_shape=jax.ShapeDtypeStruct((_B, _E, _L), jnp.float32),
    )(h1, fg_w2, gt)

    s3 = jnp.transpose(P, (1, 0, 2)).reshape(_E * _B, 1, _L)

    out = pl.pallas_call(
        _scale_body,
        grid=(_L // _CT, _B, _E),
        in_specs=[
            pl.BlockSpec((1, 1, _CT), lambda lt, b, e: (e * _B + b, 0, lt)),
            pl.BlockSpec((1, _CT, _C), lambda lt, b, e: (b, lt, 0)),
        ],
        out_specs=pl.BlockSpec((1, _CT, _C),
                               lambda lt, b, e: (e * _B + b, lt, 0)),
        out_shape=jax.ShapeDtypeStruct((_B * _E, _L, _C), jnp.float32),
    )(s3, input_features)

    return (out, P)


# speedup vs baseline: 4.1051x; 1.0399x over previous
import numpy as np
import jax
import jax.numpy as jnp
from jax.experimental import pallas as pl

_B, _L, _C, _E = 2, 2048, 2048, 8
_K = 100
_LT = 512
_CT = 1024
_HI = float(np.float32(1.0) + np.float32(1e-4))
_LO = float(np.float32(1e-4))


def _pos_code():
    half = _C // 2
    theta = 12000.0 ** (-jnp.arange(0, half, dtype=jnp.float32) / half)
    freqs = jnp.einsum('i,j->ij', jnp.arange(_L, dtype=jnp.float32), theta)
    sin = jnp.sin(freqs)
    cos = jnp.cos(freqs)
    return jnp.concatenate([cos - sin, sin + cos], axis=-1)


def _gumbel():
    u = jax.random.uniform(jax.random.key(42), (_B, _L, _E),
                           minval=1e-10, maxval=1.0)
    return -jnp.log(-jnp.log(u))


def _mm1_body(pos_ref, w1_ref, b1_ref, out_ref):
    out_ref[...] = jnp.dot(pos_ref[...], w1_ref[...],
                           preferred_element_type=jnp.float32) + b1_ref[...]


def _route_body(h1_ref, w2_ref, gt_ref, p_ref, s_ref):
    lg = jnp.dot(h1_ref[...], w2_ref[...],
                 preferred_element_type=jnp.float32)
    lt = jnp.transpose(lg[:, :_E])
    eidx = jax.lax.broadcasted_iota(jnp.int32, (_E, _L), 0)
    for b in range(_B):
        z = lt + gt_ref[b]
        zmax = jnp.max(z, axis=0, keepdims=True)
        first = jnp.min(jnp.where(z == zmax, eidx, _E), axis=0,
                        keepdims=True)
        onehot = eidx == first
        cnt = jnp.sum(onehot.astype(jnp.int32), axis=1, keepdims=True)
        small = jnp.where(cnt >= _K, 0.0, _LO).astype(jnp.float32)
        pb = jnp.where(onehot, _HI, small)
        p_ref[b] = pb
        for e in range(_E):
            s_ref[e * _B + b, 0, :] = pb[e]


def _scale_body(s_ref, x_ref, out_ref):
    st = jnp.transpose(s_ref[0])
    out_ref[...] = x_ref[...] * st[None]


def kernel(input_features, fg_w1, fg_b1, fg_w2):
    pos = _pos_code()
    gt = jnp.transpose(_gumbel(), (0, 2, 1))

    pre = pl.pallas_call(
        _mm1_body,
        grid=(_L // _LT,),
        in_specs=[
            pl.BlockSpec((_LT, _C), lambda i: (i, 0)),
            pl.BlockSpec((_C, _C), lambda i: (0, 0)),
            pl.BlockSpec((1, _C), lambda i: (0, 0)),
        ],
        out_specs=pl.BlockSpec((_LT, _C), lambda i: (i, 0)),
        out_shape=jax.ShapeDtypeStruct((_L, _C), jnp.float32),
    )(pos, fg_w1, fg_b1.reshape(1, _C))

    h1 = jax.nn.gelu(pre, approximate=False)

    P, s3 = pl.pallas_call(
        _route_body,
        in_specs=[
            pl.BlockSpec((_L, _C), lambda: (0, 0)),
            pl.BlockSpec((_C, 2 * _E), lambda: (0, 0)),
            pl.BlockSpec((_B, _E, _L), lambda: (0, 0, 0)),
        ],
        out_specs=[
            pl.BlockSpec((_B, _E, _L), lambda: (0, 0, 0)),
            pl.BlockSpec((_E * _B, 1, _L), lambda: (0, 0, 0)),
        ],
        out_shape=[
            jax.ShapeDtypeStruct((_B, _E, _L), jnp.float32),
            jax.ShapeDtypeStruct((_E * _B, 1, _L), jnp.float32),
        ],
    )(h1, fg_w2, gt)

    out = pl.pallas_call(
        _scale_body,
        grid=(_L // _CT, _B, _E),
        in_specs=[
            pl.BlockSpec((1, 1, _CT), lambda lt, b, e: (e * _B + b, 0, lt)),
            pl.BlockSpec((1, _CT, _C), lambda lt, b, e: (b, lt, 0)),
        ],
        out_specs=pl.BlockSpec((1, _CT, _C),
                               lambda lt, b, e: (e * _B + b, lt, 0)),
        out_shape=jax.ShapeDtypeStruct((_B * _E, _L, _C), jnp.float32),
    )(s3, input_features)

    return (out, P)


# speedup vs baseline: 4.9872x; 1.2149x over previous
import numpy as np
import jax
import jax.numpy as jnp
from jax.experimental import pallas as pl

_B, _L, _C, _E = 2, 2048, 2048, 8
_K = 100
_LT = 512
_CT = 1024
_HI = float(np.float32(1.0) + np.float32(1e-4))
_LO = float(np.float32(1e-4))


def _pos_code():
    half = _C // 2
    theta = 12000.0 ** (-jnp.arange(0, half, dtype=jnp.float32) / half)
    freqs = jnp.einsum('i,j->ij', jnp.arange(_L, dtype=jnp.float32), theta)
    sin = jnp.sin(freqs)
    cos = jnp.cos(freqs)
    return jnp.concatenate([cos - sin, sin + cos], axis=-1)


def _gumbel():
    u = jax.random.uniform(jax.random.key(42), (_B, _L, _E),
                           minval=1e-10, maxval=1.0)
    return -jnp.log(-jnp.log(u))


_POS = np.asarray(jax.jit(_pos_code)())
_GT = np.asarray(jax.jit(lambda: jnp.transpose(_gumbel(), (0, 2, 1)))())


def _mm1_body(pos_ref, w1_ref, b1_ref, out_ref):
    out_ref[...] = jnp.dot(pos_ref[...], w1_ref[...],
                           preferred_element_type=jnp.float32) + b1_ref[...]


def _route_body(h1_ref, w2_ref, gt_ref, p_ref, s_ref):
    lg = jnp.dot(h1_ref[...], w2_ref[...],
                 preferred_element_type=jnp.float32)
    lt = jnp.transpose(lg[:, :_E])
    eidx = jax.lax.broadcasted_iota(jnp.int32, (_E, _L), 0)
    for b in range(_B):
        z = lt + gt_ref[b]
        zmax = jnp.max(z, axis=0, keepdims=True)
        first = jnp.min(jnp.where(z == zmax, eidx, _E), axis=0,
                        keepdims=True)
        onehot = eidx == first
        cnt = jnp.sum(onehot.astype(jnp.int32), axis=1, keepdims=True)
        small = jnp.where(cnt >= _K, 0.0, _LO).astype(jnp.float32)
        pb = jnp.where(onehot, _HI, small)
        p_ref[b] = pb
        for e in range(_E):
            s_ref[e * _B + b, 0, :] = pb[e]


def _scale_body(s_ref, x_ref, out_ref):
    st = jnp.transpose(s_ref[0])
    out_ref[...] = x_ref[...] * st[None]


def kernel(input_features, fg_w1, fg_b1, fg_w2):
    pos = jnp.asarray(_POS)
    gt = jnp.asarray(_GT)

    pre = pl.pallas_call(
        _mm1_body,
        grid=(_L // _LT,),
        in_specs=[
            pl.BlockSpec((_LT, _C), lambda i: (i, 0)),
            pl.BlockSpec((_C, _C), lambda i: (0, 0)),
            pl.BlockSpec((1, _C), lambda i: (0, 0)),
        ],
        out_specs=pl.BlockSpec((_LT, _C), lambda i: (i, 0)),
        out_shape=jax.ShapeDtypeStruct((_L, _C), jnp.float32),
    )(pos, fg_w1, fg_b1.reshape(1, _C))

    h1 = jax.nn.gelu(pre, approximate=False)

    P, s3 = pl.pallas_call(
        _route_body,
        in_specs=[
            pl.BlockSpec((_L, _C), lambda: (0, 0)),
            pl.BlockSpec((_C, 2 * _E), lambda: (0, 0)),
            pl.BlockSpec((_B, _E, _L), lambda: (0, 0, 0)),
        ],
        out_specs=[
            pl.BlockSpec((_B, _E, _L), lambda: (0, 0, 0)),
            pl.BlockSpec((_E * _B, 1, _L), lambda: (0, 0, 0)),
        ],
        out_shape=[
            jax.ShapeDtypeStruct((_B, _E, _L), jnp.float32),
            jax.ShapeDtypeStruct((_E * _B, 1, _L), jnp.float32),
        ],
    )(h1, fg_w2, gt)

    out = pl.pallas_call(
        _scale_body,
        grid=(_L // _CT, _B, _E),
        in_specs=[
            pl.BlockSpec((1, 1, _CT), lambda lt, b, e: (e * _B + b, 0, lt)),
            pl.BlockSpec((1, _CT, _C), lambda lt, b, e: (b, lt, 0)),
        ],
        out_specs=pl.BlockSpec((1, _CT, _C),
                               lambda lt, b, e: (e * _B + b, lt, 0)),
        out_shape=jax.ShapeDtypeStruct((_B * _E, _L, _C), jnp.float32),
    )(s3, input_features)

    return (out, P)
